# baseline (speedup 1.0000x reference)
import functools

import jax
import jax.numpy as jnp
from jax import lax
from jax.experimental import pallas as pl
from jax.experimental.pallas import tpu as pltpu
from jax.experimental.pallas import tpu_sc as plsc

Q = 256
D = 768
K_ROWS = 100000
BLK = 2048
NB = 49
KPAD = NB * BLK
CHUNK = 128
CPB = BLK // CHUNK
NCH = KPAD // CHUNK
TOP = 5
INF = float("inf")
IBIG = 2**31 - 1

NC = 2
NS = 16
NW = NC * NS
QPW = Q // NW
LANES = 16


SLOTS = 8


def _dist_kernel(q_ref, k_ref, dist_ref, ids_ref, rcv_ref, rci_ref):
    j = pl.program_id(0)

    @pl.when(j == 0)
    def _init():
        rcv_ref[...] = jnp.full((SLOTS, Q), INF, jnp.float32)
        rci_ref[...] = jnp.full((SLOTS, Q), IBIG, jnp.int32)

    q = q_ref[...]
    kb = k_ref[...]

    ksq = jnp.sum(kb * kb, axis=1)
    qsq = jnp.sum(q * q, axis=1)
    d = jax.lax.dot_general(
        q, kb,
        dimension_numbers=(((1,), (1,)), ((), ())),
        preferred_element_type=jnp.float32,
    ) * (-2.0) + (ksq[None, :] + qsq[:, None])

    col = jax.lax.broadcasted_iota(jnp.int32, (Q, BLK), 1)
    d = jnp.where(col < K_ROWS - j * BLK, d, INF)
    dist_ref[...] = d

    dT = jax.lax.dot_general(
        kb, q,
        dimension_numbers=(((1,), (1,)), ((), ())),
        preferred_element_type=jnp.float32,
    ) * (-2.0) + (qsq[None, :] + ksq[:, None])
    row = jax.lax.broadcasted_iota(jnp.int32, (BLK, Q), 0)
    dT = jnp.where(row < K_ROWS - j * BLK, dT, INF)

    bcv = jnp.stack(
        [jnp.min(dT[c * CHUNK:(c + 1) * CHUNK, :], axis=0)
         for c in range(CPB)], axis=0)
    bci = (jax.lax.broadcasted_iota(jnp.int32, (CPB, Q), 0) + j * CPB)

    av = jnp.concatenate([rcv_ref[...], bcv], axis=0)
    ai = jnp.concatenate([rci_ref[...], bci], axis=0)
    keep_v = []
    keep_i = []
    for _ in range(TOP):
        m = jnp.min(av, axis=0)
        sel = av <= m[None, :]
        ci = jnp.min(jnp.where(sel, ai, IBIG), axis=0)
        keep_v.append(m)
        keep_i.append(ci)
        av = jnp.where(sel & (ai == ci[None, :]), INF, av)
    rcv_ref[...] = jnp.stack(
        keep_v + [jnp.full((Q,), INF, jnp.float32)] * (SLOTS - TOP), axis=0)
    rci_ref[...] = jnp.stack(
        keep_i + [jnp.full((Q,), IBIG, jnp.int32)] * (SLOTS - TOP), axis=0)

    @pl.when(j == NB - 1)
    def _emit_ids():
        ids_ref[...] = jnp.stack(
            keep_i[:TOP] + [keep_i[TOP - 1]] * (LANES - TOP), axis=0)


@functools.partial(jax.jit, static_argnames=())
def _distances_and_chunks(queries, keys):
    dists, ids = pl.pallas_call(
        _dist_kernel,
        grid=(NB,),
        in_specs=[
            pl.BlockSpec((Q, D), lambda j: (0, 0)),
            pl.BlockSpec((BLK, D), lambda j: (j, 0)),
        ],
        out_specs=[
            pl.BlockSpec((Q, BLK), lambda j: (0, j)),
            pl.BlockSpec((LANES, Q), lambda j: (0, 0)),
        ],
        out_shape=[
            jax.ShapeDtypeStruct((Q, KPAD), jnp.float32),
            jax.ShapeDtypeStruct((LANES, Q), jnp.int32),
        ],
        scratch_shapes=[
            pltpu.VMEM((SLOTS, Q), jnp.float32),
            pltpu.VMEM((SLOTS, Q), jnp.int32),
        ],
        compiler_params=pltpu.CompilerParams(
            dimension_semantics=("arbitrary",),
        ),
    )(queries, keys)
    return dists, ids


def _sc_topk_kernel(ids_hbm, dist_hbm, vals_hbm, idx_hbm,
                    ids_v, cand_v, ov_v, oi_v, sem):
    wid = lax.axis_index("s") * NC + lax.axis_index("c")
    pltpu.sync_copy(ids_hbm, ids_v)
    lane = lax.iota(jnp.int32, LANES)

    def do_query(qi, _):
        q = wid * QPW + qi
        civ = plsc.load_gather(
            ids_v, [lane, jnp.full((LANES,), q, jnp.int32)])
        iv = q * NCH + civ
        pltpu.async_copy(dist_hbm.at[iv], cand_v, sem).wait()

        pv, pi = jnp.float32(-INF), jnp.int32(-1)
        ov = jnp.full((LANES,), INF, jnp.float32)
        oi = jnp.full((LANES,), IBIG, jnp.int32)
        for t_out in range(TOP):
            accv = jnp.full((LANES,), INF, jnp.float32)
            acci = jnp.full((LANES,), IBIG, jnp.int32)
            for t in range(TOP):
                cbase = civ[t] * CHUNK

                def scan_vreg(r, c2, t=t, cbase=cbase):
                    av, ai = c2
                    v = cand_v[t, pl.ds(r * LANES, LANES)]
                    gi = cbase + r * LANES + lane
                    ok = (v > pv) | ((v == pv) & (gi > pi))
                    vv = jnp.where(ok, v, INF)
                    take = (vv < av) | ((vv == av) & (gi < ai))
                    return (jnp.where(take, vv, av),
                            jnp.where(take, gi, ai))

                accv, acci = lax.fori_loop(0, CHUNK // LANES, scan_vreg,
                                           (accv, acci))
            sk, sv = plsc.sort_key_val(accv, acci)
            m = sk[0]
            ii = sv[0]
            ov = jnp.where(lane == t_out, m, ov)
            oi = jnp.where(lane == t_out, ii, oi)
            pv, pi = m, ii
        ov_v[...] = ov
        oi_v[...] = oi
        pltpu.sync_copy(ov_v, vals_hbm.at[q])
        pltpu.sync_copy(oi_v, idx_hbm.at[q])
        return 0

    lax.fori_loop(0, QPW, do_query, 0)


@functools.partial(jax.jit, static_argnames=())
def _sc_topk(ids, dists2):
    f = functools.partial(
        pl.kernel,
        mesh=plsc.VectorSubcoreMesh(core_axis_name="c", subcore_axis_name="s"),
        out_type=[
            jax.ShapeDtypeStruct((Q, LANES), jnp.float32),
            jax.ShapeDtypeStruct((Q, LANES), jnp.int32),
        ],
        scratch_types=[
            pltpu.VMEM((LANES, Q), jnp.int32),
            pltpu.VMEM((LANES, CHUNK), jnp.float32),
            pltpu.VMEM((LANES,), jnp.float32),
            pltpu.VMEM((LANES,), jnp.int32),
            pltpu.SemaphoreType.DMA,
        ],
        compiler_params=pltpu.CompilerParams(needs_layout_passes=False),
    )(_sc_topk_kernel)
    return f(ids, dists2)


def kernel(queries, keys, k):
    del k
    dists, ids = _distances_and_chunks(queries, keys)
    dists2 = dists.reshape(Q * NCH, CHUNK)
    vals16, idx16 = _sc_topk(ids, dists2)
    return vals16[:, :TOP], idx16[:, :TOP]


# speedup vs baseline: 31.5736x; 31.5736x over previous
import functools

import jax
import jax.numpy as jnp
from jax import lax
from jax.experimental import pallas as pl
from jax.experimental.pallas import tpu as pltpu
from jax.experimental.pallas import tpu_sc as plsc

Q = 256
D = 768
K_ROWS = 100000
BLK = 2000
NB = K_ROWS // BLK
NBP = 64
TOP = 5
LANES = 16
INF = float("inf")
IBIG = 2**31 - 1

NC = 2
NS = 16
NW = NC * NS
QPW = Q // NW


def _block_topk_kernel(q_ref, k_ref, cv_ref):
    j = pl.program_id(0)
    q = q_ref[...]
    kb = k_ref[...]

    ksq = jnp.sum(kb * kb, axis=1)
    qsq = jnp.sum(q * q, axis=1)
    d = jax.lax.dot_general(
        q, kb,
        dimension_numbers=(((1,), (1,)), ((), ())),
        preferred_element_type=jnp.float32,
    ) * (-2.0) + ksq[None, :]

    col = jax.lax.broadcasted_iota(jnp.int32, (Q, BLK), 1)
    vlist = []
    ilist = []
    for _ in range(TOP):
        m = jnp.min(d, axis=1)
        a = jnp.min(jnp.where(d <= m[:, None], col, IBIG), axis=1)
        vlist.append(m)
        ilist.append(a + j * BLK)
        d = jnp.where(col == a[:, None], INF, d)
    for _ in range(LANES - TOP):
        vlist.append(jnp.full((Q,), INF, jnp.float32))
        ilist.append(jnp.full((Q,), IBIG, jnp.int32))
    vi = jnp.concatenate(
        [jnp.stack(vlist, axis=1) + qsq[:, None],
         jax.lax.bitcast_convert_type(jnp.stack(ilist, axis=1), jnp.float32),
         jnp.full((Q, 128 - 2 * LANES), INF, jnp.float32)], axis=1)
    cv_ref[0] = vi


@functools.partial(jax.jit, static_argnames=())
def _block_candidates(queries, keys):
    cv, = pl.pallas_call(
        _block_topk_kernel,
        grid=(NB,),
        in_specs=[
            pl.BlockSpec((Q, D), lambda j: (0, 0)),
            pl.BlockSpec((BLK, D), lambda j: (j, 0)),
        ],
        out_specs=[
            pl.BlockSpec((1, Q, 128), lambda j: (j, 0, 0)),
        ],
        out_shape=[
            jax.ShapeDtypeStruct((NB, Q, 128), jnp.float32),
        ],
        compiler_params=pltpu.CompilerParams(
            dimension_semantics=("arbitrary",),
        ),
    )(queries, keys)
    return cv


def _sc_merge_kernel(cv_hbm, vals_hbm, idx_hbm,
                     iv_v, cand_v, ov_v, oi_v, sem):
    wid = lax.axis_index("s") * NC + lax.axis_index("c")
    lane = lax.iota(jnp.int32, LANES)

    def do_query(qi, _):
        q = wid * QPW + qi
        for b in range(NBP // LANES):
            r = jnp.minimum(b * LANES + lane, NB - 1)
            iv_v[pl.ds(b * LANES, LANES)] = r * Q + q
        pltpu.async_copy(cv_hbm.at[iv_v], cand_v, sem).wait()

        pv, pi = jnp.float32(-INF), jnp.int32(-1)
        ov = jnp.full((LANES,), INF, jnp.float32)
        oi = jnp.full((LANES,), IBIG, jnp.int32)
        for t_out in range(TOP):
            accv = jnp.full((LANES,), INF, jnp.float32)
            acci = jnp.full((LANES,), IBIG, jnp.int32)

            def scan_row(t, c2):
                av, ai = c2
                v = cand_v[t, pl.ds(0, LANES)]
                gi = plsc.bitcast(cand_v[t, pl.ds(LANES, LANES)], jnp.int32)
                ok = (v > pv) | ((v == pv) & (gi > pi))
                vv = jnp.where(ok, v, INF)
                take = (vv < av) | ((vv == av) & (gi < ai))
                return (jnp.where(take, vv, av),
                        jnp.where(take, gi, ai))

            accv, acci = lax.fori_loop(0, NBP, scan_row, (accv, acci))
            sk, sv = plsc.sort_key_val(accv, acci)
            m = sk[0]
            ii = sv[0]
            ov = jnp.where(lane == t_out, m, ov)
            oi = jnp.where(lane == t_out, ii, oi)
            pv, pi = m, ii
        ov_v[...] = ov
        oi_v[...] = oi
        pltpu.sync_copy(ov_v, vals_hbm.at[q])
        pltpu.sync_copy(oi_v, idx_hbm.at[q])
        return 0

    lax.fori_loop(0, QPW, do_query, 0)


@functools.partial(jax.jit, static_argnames=())
def _sc_merge(cv2):
    f = functools.partial(
        pl.kernel,
        mesh=plsc.VectorSubcoreMesh(core_axis_name="c", subcore_axis_name="s"),
        out_type=[
            jax.ShapeDtypeStruct((Q, LANES), jnp.float32),
            jax.ShapeDtypeStruct((Q, LANES), jnp.int32),
        ],
        scratch_types=[
            pltpu.VMEM((NBP,), jnp.int32),
            pltpu.VMEM((NBP, 128), jnp.float32),
            pltpu.VMEM((LANES,), jnp.float32),
            pltpu.VMEM((LANES,), jnp.int32),
            pltpu.SemaphoreType.DMA,
        ],
        compiler_params=pltpu.CompilerParams(needs_layout_passes=False),
    )(_sc_merge_kernel)
    return f(cv2)


def kernel(queries, keys, k):
    del k
    cv = _block_candidates(queries, keys)
    vals16, idx16 = _sc_merge(cv.reshape(NB * Q, 128))
    return vals16[:, :TOP], idx16[:, :TOP]
